# baseline (speedup 1.0000x reference)
import functools

import jax
import jax.numpy as jnp
from jax import lax
from jax.experimental import pallas as pl
from jax.experimental.pallas import tpu as pltpu
from jax.experimental.pallas import tpu_sc as plsc

D_MODEL = 768
HID = 4 * D_MODEL
N_EXP = 64
CAP = 64
TOKENS = 4096


N_HALF = 2
HID_BLK = HID // N_HALF


_LANES = 16
_NCHUNK = TOKENS // _LANES
_LIST = 96


def _dispatch_body(ix_hbm, sel_hbm, ix_v, list0_v, list1_v, tmp_v):
    cid = lax.axis_index("c")
    sid = lax.axis_index("s")
    wid = sid * 2 + cid
    pltpu.sync_copy(ix_hbm, ix_v)
    neg1 = jnp.full((_LANES,), -1, jnp.int32)
    for j in range(_LIST // _LANES):
        list0_v[pl.ds(j * _LANES, _LANES)] = neg1
        list1_v[pl.ds(j * _LANES, _LANES)] = neg1
    lane = lax.iota(jnp.int32, _LANES)

    gdn = lax.GatherDimensionNumbers(
        offset_dims=(), collapsed_slice_dims=(0,), start_index_map=(0,))

    def _lane_gather(x, src):
        return lax.gather(x, src[:, None], gdn, slice_sizes=(1,),
                          mode=lax.GatherScatterMode.PROMISE_IN_BOUNDS)

    def _prefix_incl(m):
        pref = m
        for sh in (1, 2, 4, 8):
            src = jnp.maximum(lane - sh, 0)
            keep = jnp.minimum(1, jnp.maximum(lane - sh + 1, 0))
            pref = pref + _lane_gather(pref, src) * keep
        return pref

    e0 = wid * 2
    e1 = wid * 2 + 1

    def body(i, carry):
        c0, c1 = carry
        v = ix_v[pl.ds(i * _LANES, _LANES)]
        tok = lane + i * _LANES
        m0 = jnp.maximum(0, 1 - jnp.abs(v - e0))
        m1 = jnp.maximum(0, 1 - jnp.abs(v - e1))
        pref01 = plsc.cumsum(m0 + m1 * 65536)
        pref0 = jnp.bitwise_and(pref01, 65535)
        pref1 = lax.shift_right_logical(pref01, 16)
        pos0 = c0 + pref0 - m0
        pos1 = c1 + pref1 - m1
        idx0 = jnp.minimum(pos0, _LIST - 2) * m0 + (_LIST - 1) * (1 - m0)
        idx1 = jnp.minimum(pos1, _LIST - 2) * m1 + (_LIST - 1) * (1 - m1)
        plsc.store_scatter(list0_v, [idx0], tok)
        plsc.store_scatter(list1_v, [idx1], tok)
        return (c0 + pref0[_LANES - 1], c1 + pref1[_LANES - 1])

    lax.fori_loop(0, _NCHUNK, body, (jnp.int32(0), jnp.int32(0)),
                  unroll=4)
    pltpu.sync_copy(list0_v.at[pl.ds(0, CAP)],
                    sel_hbm.at[pl.ds(wid * 2 * CAP, CAP)])
    pltpu.sync_copy(list1_v.at[pl.ds(0, CAP)],
                    sel_hbm.at[pl.ds((wid * 2 + 1) * CAP, CAP)])


_dispatch_sc = functools.partial(
    pl.kernel,
    mesh=plsc.VectorSubcoreMesh(core_axis_name="c", subcore_axis_name="s"),
    compiler_params=pltpu.CompilerParams(needs_layout_passes=False),
    out_type=jax.ShapeDtypeStruct((N_EXP * CAP,), jnp.int32),
    scratch_types=[
        pltpu.VMEM((TOKENS,), jnp.int32),
        pltpu.VMEM((_LIST,), jnp.int32),
        pltpu.VMEM((_LIST,), jnp.int32),
        pltpu.VMEM((_LANES,), jnp.int32),
    ],
)(_dispatch_body)


def _ffn_body(sel_ref, x_ref, w1_ref, b1_ref, w2_ref, b2_ref, out_ref,
              h_scr, y_scr):
    e = pl.program_id(0)

    @pl.when(e == 0)
    def _zero():
        out_ref[...] = jnp.zeros_like(out_ref)

    def gather_body(c, carry):
        tok = jnp.maximum(sel_ref[0, 0, c], 0)
        h_scr[pl.ds(c, 1), :] = x_ref[pl.ds(tok, 1), :]
        return carry

    jax.lax.fori_loop(0, CAP, gather_body, 0, unroll=8)

    h = h_scr[...].astype(jnp.bfloat16)
    hid = jnp.dot(h, w1_ref[0].astype(jnp.bfloat16),
                  preferred_element_type=jnp.float32)
    hid = jnp.maximum(hid + b1_ref[0], 0.0).astype(jnp.bfloat16)
    y = jnp.dot(hid, w2_ref[0].astype(jnp.bfloat16),
                preferred_element_type=jnp.float32)
    y_scr[...] = y + b2_ref[0]

    def scatter_body(c, carry):
        tok = sel_ref[0, 0, c]

        @pl.when(tok >= 0)
        def _():
            out_ref[pl.ds(tok, 1), :] = y_scr[pl.ds(c, 1), :]

        return carry

    jax.lax.fori_loop(0, CAP, scatter_body, 0, unroll=8)


def _ffn_call(sel, xf, W1, b1, W2, b2):
    return pl.pallas_call(
        _ffn_body,
        grid=(N_EXP,),
        in_specs=[
            pl.BlockSpec((1, 1, CAP), lambda e: (e, 0, 0),
                         memory_space=pltpu.SMEM),
            pl.BlockSpec((TOKENS, D_MODEL), lambda e: (0, 0)),
            pl.BlockSpec((1, D_MODEL, HID), lambda e: (e, 0, 0)),
            pl.BlockSpec((1, 1, HID), lambda e: (e, 0, 0)),
            pl.BlockSpec((1, HID, D_MODEL), lambda e: (e, 0, 0)),
            pl.BlockSpec((1, 1, D_MODEL), lambda e: (e, 0, 0)),
        ],
        out_specs=pl.BlockSpec((TOKENS, D_MODEL), lambda e: (0, 0)),
        out_shape=jax.ShapeDtypeStruct((TOKENS, D_MODEL), jnp.float32),
        scratch_shapes=[
            pltpu.VMEM((CAP, D_MODEL), jnp.float32),
            pltpu.VMEM((CAP, D_MODEL), jnp.float32),
        ],
        compiler_params=pltpu.CompilerParams(
            dimension_semantics=("arbitrary",),
            vmem_limit_bytes=100 * 1024 * 1024,
        ),
    )(sel, xf, W1, b1, W2, b2)


def kernel(x, noise, Wl, bl, Wn, bn, W1, b1, W2, b2):
    Bsz, Tlen, d = x.shape
    xf = x.reshape(-1, d)

    Wcat = jnp.concatenate([Wl, Wn], axis=1)
    z = x @ Wcat
    logits = z[..., :N_EXP] + bl
    scale = jax.nn.softplus(z[..., N_EXP:] + bn)
    noisy = (logits + noise * scale).reshape(-1, N_EXP)
    ix = jnp.argmax(noisy, axis=-1).astype(jnp.int32)

    sel = _dispatch_sc(ix).reshape(N_EXP, 1, CAP)

    y = _ffn_call(sel, xf, W1, b1.reshape(N_EXP, 1, HID), W2,
                  b2.reshape(N_EXP, 1, D_MODEL))
    return y.reshape(Bsz, Tlen, d)


# speedup vs baseline: 1.0224x; 1.0224x over previous
import functools

import jax
import jax.numpy as jnp
from jax import lax
from jax.experimental import pallas as pl
from jax.experimental.pallas import tpu as pltpu
from jax.experimental.pallas import tpu_sc as plsc

D_MODEL = 768
HID = 4 * D_MODEL
N_EXP = 64
CAP = 64
TOKENS = 4096


N_HALF = 2
HID_BLK = HID // N_HALF


_LANES = 16
_NCHUNK = TOKENS // _LANES
_LIST = 96


def _dispatch_body(ix_hbm, sel_hbm, ix_v, list0_v, list1_v):
    cid = lax.axis_index("c")
    sid = lax.axis_index("s")
    wid = sid * 2 + cid
    pltpu.sync_copy(ix_hbm, ix_v)
    neg1 = jnp.full((_LANES,), -1, jnp.int32)
    for j in range(_LIST // _LANES):
        list0_v[pl.ds(j * _LANES, _LANES)] = neg1
        list1_v[pl.ds(j * _LANES, _LANES)] = neg1
    lane = lax.iota(jnp.int32, _LANES)

    gdn = lax.GatherDimensionNumbers(
        offset_dims=(), collapsed_slice_dims=(0,), start_index_map=(0,))

    def _lane_gather(x, src):
        return lax.gather(x, src[:, None], gdn, slice_sizes=(1,),
                          mode=lax.GatherScatterMode.PROMISE_IN_BOUNDS)

    def _prefix_incl(m):
        pref = m
        for sh in (1, 2, 4, 8):
            src = jnp.maximum(lane - sh, 0)
            keep = jnp.minimum(1, jnp.maximum(lane - sh + 1, 0))
            pref = pref + _lane_gather(pref, src) * keep
        return pref

    e0 = wid * 2
    e1 = wid * 2 + 1

    def body(i, carry):
        c0, c1 = carry
        v = ix_v[pl.ds(i * _LANES, _LANES)]
        tok = lane + i * _LANES
        m0 = jnp.maximum(0, 1 - jnp.abs(v - e0))
        m1 = jnp.maximum(0, 1 - jnp.abs(v - e1))
        pref01 = _prefix_incl(m0 + m1 * 65536)
        pref0 = jnp.bitwise_and(pref01, 65535)
        pref1 = lax.shift_right_logical(pref01, 16)
        pos0 = c0 + pref0 - m0
        pos1 = c1 + pref1 - m1
        idx0 = jnp.minimum(pos0, _LIST - 2) * m0 + (_LIST - 1) * (1 - m0)
        idx1 = jnp.minimum(pos1, _LIST - 2) * m1 + (_LIST - 1) * (1 - m1)
        plsc.store_scatter(list0_v, [idx0], tok)
        plsc.store_scatter(list1_v, [idx1], tok)
        return (c0 + pref0[_LANES - 1], c1 + pref1[_LANES - 1])

    lax.fori_loop(0, _NCHUNK, body, (jnp.int32(0), jnp.int32(0)),
                  unroll=8)
    pltpu.sync_copy(list0_v.at[pl.ds(0, CAP)],
                    sel_hbm.at[pl.ds(wid * 2 * CAP, CAP)])
    pltpu.sync_copy(list1_v.at[pl.ds(0, CAP)],
                    sel_hbm.at[pl.ds((wid * 2 + 1) * CAP, CAP)])


_dispatch_sc = functools.partial(
    pl.kernel,
    mesh=plsc.VectorSubcoreMesh(core_axis_name="c", subcore_axis_name="s"),
    compiler_params=pltpu.CompilerParams(needs_layout_passes=False),
    out_type=jax.ShapeDtypeStruct((N_EXP * CAP,), jnp.int32),
    scratch_types=[
        pltpu.VMEM((TOKENS,), jnp.int32),
        pltpu.VMEM((_LIST,), jnp.int32),
        pltpu.VMEM((_LIST,), jnp.int32),
    ],
)(_dispatch_body)


def _ffn_body(sel_ref, x_ref, w1_ref, b1_ref, w2_ref, b2_ref, out_ref,
              h_scr, y_scr):
    e = pl.program_id(0)

    @pl.when(e == 0)
    def _zero():
        out_ref[...] = jnp.zeros_like(out_ref)

    def gather_body(c, carry):
        tok = jnp.maximum(sel_ref[0, 0, c], 0)
        h_scr[pl.ds(c, 1), :] = x_ref[pl.ds(tok, 1), :]
        return carry

    jax.lax.fori_loop(0, CAP, gather_body, 0, unroll=8)

    h = h_scr[...].astype(jnp.bfloat16)
    hid = jnp.dot(h, w1_ref[0].astype(jnp.bfloat16),
                  preferred_element_type=jnp.float32)
    hid = jnp.maximum(hid + b1_ref[0], 0.0).astype(jnp.bfloat16)
    y = jnp.dot(hid, w2_ref[0].astype(jnp.bfloat16),
                preferred_element_type=jnp.float32)
    y_scr[...] = y + b2_ref[0]

    def scatter_body(c, carry):
        tok = sel_ref[0, 0, c]

        @pl.when(tok >= 0)
        def _():
            out_ref[pl.ds(tok, 1), :] = y_scr[pl.ds(c, 1), :]

        return carry

    jax.lax.fori_loop(0, CAP, scatter_body, 0, unroll=8)


def _ffn_call(sel, xf, W1, b1, W2, b2):
    return pl.pallas_call(
        _ffn_body,
        grid=(N_EXP,),
        in_specs=[
            pl.BlockSpec((1, 1, CAP), lambda e: (e, 0, 0),
                         memory_space=pltpu.SMEM),
            pl.BlockSpec((TOKENS, D_MODEL), lambda e: (0, 0)),
            pl.BlockSpec((1, D_MODEL, HID), lambda e: (e, 0, 0)),
            pl.BlockSpec((1, 1, HID), lambda e: (e, 0, 0)),
            pl.BlockSpec((1, HID, D_MODEL), lambda e: (e, 0, 0)),
            pl.BlockSpec((1, 1, D_MODEL), lambda e: (e, 0, 0)),
        ],
        out_specs=pl.BlockSpec((TOKENS, D_MODEL), lambda e: (0, 0)),
        out_shape=jax.ShapeDtypeStruct((TOKENS, D_MODEL), jnp.float32),
        scratch_shapes=[
            pltpu.VMEM((CAP, D_MODEL), jnp.float32),
            pltpu.VMEM((CAP, D_MODEL), jnp.float32),
        ],
        compiler_params=pltpu.CompilerParams(
            dimension_semantics=("arbitrary",),
            vmem_limit_bytes=100 * 1024 * 1024,
        ),
    )(sel, xf, W1, b1, W2, b2)


def kernel(x, noise, Wl, bl, Wn, bn, W1, b1, W2, b2):
    Bsz, Tlen, d = x.shape
    xf = x.reshape(-1, d)

    Wcat = jnp.concatenate([Wl, Wn], axis=1)
    z = x @ Wcat
    logits = z[..., :N_EXP] + bl
    scale = jax.nn.softplus(z[..., N_EXP:] + bn)
    noisy = (logits + noise * scale).reshape(-1, N_EXP)
    ix = jnp.argmax(noisy, axis=-1).astype(jnp.int32)

    sel = _dispatch_sc(ix).reshape(N_EXP, 1, CAP)

    y = _ffn_call(sel, xf, W1, b1.reshape(N_EXP, 1, HID), W2,
                  b2.reshape(N_EXP, 1, D_MODEL))
    return y.reshape(Bsz, Tlen, d)


# speedup vs baseline: 1.0243x; 1.0019x over previous
import functools

import jax
import jax.numpy as jnp
from jax import lax
from jax.experimental import pallas as pl
from jax.experimental.pallas import tpu as pltpu
from jax.experimental.pallas import tpu_sc as plsc

D_MODEL = 768
HID = 4 * D_MODEL
N_EXP = 64
CAP = 64
TOKENS = 4096


N_HALF = 2
HID_BLK = HID // N_HALF


_LANES = 16
_NCHUNK = TOKENS // _LANES
_LIST = 96


def _dispatch_body(ix_hbm, sel_hbm, ix_v, list0_v, list1_v):
    cid = lax.axis_index("c")
    sid = lax.axis_index("s")
    wid = sid * 2 + cid
    pltpu.sync_copy(ix_hbm, ix_v)
    neg1 = jnp.full((_LANES,), -1, jnp.int32)
    for j in range(_LIST // _LANES):
        list0_v[pl.ds(j * _LANES, _LANES)] = neg1
        list1_v[pl.ds(j * _LANES, _LANES)] = neg1
    lane = lax.iota(jnp.int32, _LANES)

    gdn = lax.GatherDimensionNumbers(
        offset_dims=(), collapsed_slice_dims=(0,), start_index_map=(0,))

    def _lane_gather(x, src):
        return lax.gather(x, src[:, None], gdn, slice_sizes=(1,),
                          mode=lax.GatherScatterMode.PROMISE_IN_BOUNDS)

    def _prefix_incl(m):
        pref = m
        for sh in (1, 2, 4, 8):
            src = jnp.maximum(lane - sh, 0)
            keep = jnp.minimum(1, jnp.maximum(lane - sh + 1, 0))
            pref = pref + _lane_gather(pref, src) * keep
        return pref

    e0 = wid * 2
    e1 = wid * 2 + 1

    def body(i, carry):
        c0, c1 = carry
        v = ix_v[pl.ds(i * _LANES, _LANES)]
        tok = lane + i * _LANES
        m0 = jnp.maximum(0, 1 - jnp.abs(v - e0))
        m1 = jnp.maximum(0, 1 - jnp.abs(v - e1))
        pref01 = _prefix_incl(m0 + m1 * 65536)
        pref0 = jnp.bitwise_and(pref01, 65535)
        pref1 = lax.shift_right_logical(pref01, 16)
        pos0 = c0 + pref0 - m0
        pos1 = c1 + pref1 - m1
        idx0 = jnp.minimum(pos0, _LIST - 2) * m0 + (_LIST - 1) * (1 - m0)
        idx1 = jnp.minimum(pos1, _LIST - 2) * m1 + (_LIST - 1) * (1 - m1)
        plsc.store_scatter(list0_v, [idx0], tok)
        plsc.store_scatter(list1_v, [idx1], tok)
        return (c0 + pref0[_LANES - 1], c1 + pref1[_LANES - 1])

    lax.fori_loop(0, _NCHUNK, body, (jnp.int32(0), jnp.int32(0)),
                  unroll=8)
    pltpu.sync_copy(list0_v.at[pl.ds(0, CAP)],
                    sel_hbm.at[pl.ds(wid * 2 * CAP, CAP)])
    pltpu.sync_copy(list1_v.at[pl.ds(0, CAP)],
                    sel_hbm.at[pl.ds((wid * 2 + 1) * CAP, CAP)])


_dispatch_sc = functools.partial(
    pl.kernel,
    mesh=plsc.VectorSubcoreMesh(core_axis_name="c", subcore_axis_name="s"),
    compiler_params=pltpu.CompilerParams(needs_layout_passes=False),
    out_type=jax.ShapeDtypeStruct((N_EXP * CAP,), jnp.int32),
    scratch_types=[
        pltpu.VMEM((TOKENS,), jnp.int32),
        pltpu.VMEM((_LIST,), jnp.int32),
        pltpu.VMEM((_LIST,), jnp.int32),
    ],
)(_dispatch_body)


def _ffn_body(sel_ref, x_ref, w1_ref, b1_ref, w2_ref, b2_ref, out_ref,
              h_scr, y_scr):
    e = pl.program_id(0)

    @pl.when(e == 0)
    def _zero():
        out_ref[...] = jnp.zeros_like(out_ref)

    def gather_body(c, carry):
        tok = jnp.maximum(sel_ref[0, 0, c], 0)
        h_scr[pl.ds(c, 1), :] = x_ref[pl.ds(tok, 1), :]
        return carry

    jax.lax.fori_loop(0, CAP, gather_body, 0, unroll=16)

    h = h_scr[...].astype(jnp.bfloat16)
    hid = jnp.dot(h, w1_ref[0].astype(jnp.bfloat16),
                  preferred_element_type=jnp.float32)
    hid = jnp.maximum(hid + b1_ref[0], 0.0).astype(jnp.bfloat16)
    y = jnp.dot(hid, w2_ref[0].astype(jnp.bfloat16),
                preferred_element_type=jnp.float32)
    y_scr[...] = y + b2_ref[0]

    def scatter_body(c, carry):
        tok = sel_ref[0, 0, c]

        @pl.when(tok >= 0)
        def _():
            out_ref[pl.ds(tok, 1), :] = y_scr[pl.ds(c, 1), :]

        return carry

    jax.lax.fori_loop(0, CAP, scatter_body, 0, unroll=16)


def _ffn_call(sel, xf, W1, b1, W2, b2):
    return pl.pallas_call(
        _ffn_body,
        grid=(N_EXP,),
        in_specs=[
            pl.BlockSpec((1, 1, CAP), lambda e: (e, 0, 0),
                         memory_space=pltpu.SMEM),
            pl.BlockSpec((TOKENS, D_MODEL), lambda e: (0, 0)),
            pl.BlockSpec((1, D_MODEL, HID), lambda e: (e, 0, 0)),
            pl.BlockSpec((1, 1, HID), lambda e: (e, 0, 0)),
            pl.BlockSpec((1, HID, D_MODEL), lambda e: (e, 0, 0)),
            pl.BlockSpec((1, 1, D_MODEL), lambda e: (e, 0, 0)),
        ],
        out_specs=pl.BlockSpec((TOKENS, D_MODEL), lambda e: (0, 0)),
        out_shape=jax.ShapeDtypeStruct((TOKENS, D_MODEL), jnp.float32),
        scratch_shapes=[
            pltpu.VMEM((CAP, D_MODEL), jnp.float32),
            pltpu.VMEM((CAP, D_MODEL), jnp.float32),
        ],
        compiler_params=pltpu.CompilerParams(
            dimension_semantics=("arbitrary",),
            vmem_limit_bytes=100 * 1024 * 1024,
        ),
    )(sel, xf, W1, b1, W2, b2)


def kernel(x, noise, Wl, bl, Wn, bn, W1, b1, W2, b2):
    Bsz, Tlen, d = x.shape
    xf = x.reshape(-1, d)

    Wcat = jnp.concatenate([Wl, Wn], axis=1)
    z = x @ Wcat
    logits = z[..., :N_EXP] + bl
    scale = jax.nn.softplus(z[..., N_EXP:] + bn)
    noisy = (logits + noise * scale).reshape(-1, N_EXP)
    ix = jnp.argmax(noisy, axis=-1).astype(jnp.int32)

    sel = _dispatch_sc(ix).reshape(N_EXP, 1, CAP)

    y = _ffn_call(sel, xf, W1, b1.reshape(N_EXP, 1, HID), W2,
                  b2.reshape(N_EXP, 1, D_MODEL))
    return y.reshape(Bsz, Tlen, d)
